# baseline (speedup 1.0000x reference)
import functools

import jax
import jax.numpy as jnp
from jax.experimental import pallas as pl
from jax.experimental.pallas import tpu as pltpu
from jax.experimental.pallas import tpu_sc as plsc

N_C, N_G, N_P = 512, 256, 128
D_IN, HID = 512, 256
N_NODES = N_C + N_G + N_P
N_EDGES = 131072
EPS = 0.1
NN2 = N_NODES * N_NODES

_SC_CORES = 2
_SC_SUBCORES = 16
_NW = _SC_CORES * _SC_SUBCORES
_EPW = N_EDGES // _NW
_CH = _EPW // 128
_ZPW = NN2 // _SC_SUBCORES


def _edge_hist_body(edge_hbm, out_hbm, src_v, dst_v, idx_v, ones_v,
                    zbuf_a, zbuf_b, stage_a, stage_b,
                    sem, sem_i0, sem_i1, sem_o0, sem_o1, csh):
    c = jax.lax.axis_index("c")
    s = jax.lax.axis_index("s")
    wid = s * _SC_CORES + c
    base = wid * _EPW
    cp_s = pltpu.async_copy(edge_hbm.at[0, pl.ds(base, _EPW)], src_v, sem)
    cp_d = pltpu.async_copy(edge_hbm.at[1, pl.ds(base, _EPW)], dst_v, sem)
    for k in range(_ZBUF // 16):
        zbuf_a[pl.ds(k * 16, 16)] = jnp.zeros((16,), jnp.float32)
    zcopies = [
        pltpu.async_copy(zbuf_a, csh.at[pl.ds(s * _ZPW + q * _ZBUF, _ZBUF)],
                         sem_i0)
        for q in range(_ZPW // _ZBUF)
    ]
    for k in range(8):
        ones_v[pl.ds(k * 16, 16)] = jnp.ones((16,), jnp.float32)
    cp_s.wait()
    cp_d.wait()
    for j in range(_CH):
        for k in range(8):
            o = (j * 8 + k) * 16
            srcs = src_v[pl.ds(o, 16)]
            dsts = dst_v[pl.ds(o, 16)]
            idx_v[j, pl.ds(k * 16, 16)] = (
                (srcs >> 7) * (N_NODES * 128) + dsts * 128 + (srcs & 127))
    for cp in zcopies:
        cp.wait()
    plsc.subcore_barrier()
    copies = [
        pltpu.async_copy(ones_v, csh.at[idx_v.at[j]], sem, add=True)
        for j in range(_CH)
    ]
    for cp in copies:
        cp.wait()
    plsc.subcore_barrier()
    crows = _ZBUF // 128
    nch = _ZPW // _ZBUF
    zbufs = [zbuf_a, zbuf_b]
    stages = [stage_a, stage_b]
    sem_in = [sem_i0, sem_i1]
    sem_out = [sem_o0, sem_o1]
    in_cp = [None, None]
    out_cp = [None, None]

    def _chunk_src(q):
        return csh.at[pl.ds(s * _ZPW + q * _ZBUF, _ZBUF)]

    in_cp[0] = pltpu.async_copy(_chunk_src(0), zbuf_a, sem_i0)
    for q in range(nch):
        b = q & 1
        in_cp[b].wait()
        if q + 1 < nch:
            in_cp[1 - b] = pltpu.async_copy(_chunk_src(q + 1), zbufs[1 - b],
                                            sem_in[1 - b])
        if out_cp[b] is not None:
            out_cp[b].wait()
        zb = zbufs[b]
        st = stages[b]

        def _row(r, carry):
            for k in range(8):
                st[r, pl.ds(k * 16, 16)] = zb[pl.ds(r * 128 + k * 16, 16)]
            return carry

        jax.lax.fori_loop(0, crows, _row, 0)
        out_cp[b] = pltpu.async_copy(
            st, out_hbm.at[c, pl.ds(s * (_ZPW // 128) + q * crows, crows), :],
            sem_out[b])
    out_cp[0].wait()
    out_cp[1].wait()


_ZBUF = 7168


@functools.cache
def _edge_hist():
    return pl.kernel(
        _edge_hist_body,
        out_type=jax.ShapeDtypeStruct((_SC_CORES, NN2 // 128, 128),
                                      jnp.float32),
        mesh=plsc.VectorSubcoreMesh(core_axis_name="c", subcore_axis_name="s",
                                    num_cores=_SC_CORES,
                                    num_subcores=_SC_SUBCORES),
        scratch_types=[
            pltpu.VMEM((_EPW,), jnp.int32),
            pltpu.VMEM((_EPW,), jnp.int32),
            pltpu.VMEM((_CH, 128), jnp.int32),
            pltpu.VMEM((128,), jnp.float32),
            pltpu.VMEM((_ZBUF,), jnp.float32),
            pltpu.VMEM((_ZBUF,), jnp.float32),
            pltpu.VMEM((_ZBUF // 128, 128), jnp.float32),
            pltpu.VMEM((_ZBUF // 128, 128), jnp.float32),
            pltpu.SemaphoreType.DMA,
            pltpu.SemaphoreType.DMA,
            pltpu.SemaphoreType.DMA,
            pltpu.SemaphoreType.DMA,
            pltpu.SemaphoreType.DMA,
            pltpu.VMEM_SHARED((NN2,), jnp.float32),
        ],
    )


def _kl_terms(log_pred, target_logits):
    m = jnp.max(target_logits, axis=-1, keepdims=True)
    e = jnp.exp(target_logits - m)
    ssum = jnp.sum(e, axis=-1, keepdims=True)
    t = e / ssum
    log_t = (target_logits - m) - jnp.log(ssum)
    return jnp.mean(t * (log_t - log_pred))


def _log_softmax(z):
    m = jnp.max(z, axis=-1, keepdims=True)
    zs = z - m
    return zs - jnp.log(jnp.sum(jnp.exp(zs), axis=-1, keepdims=True))


def _stage_a_body(x_ref, win_ref, bin_ref, c2_ref, wmsg_ref, bmsg_ref,
                  trna_ref, tatac_ref, lab_ref, wdec_ref, bdec_ref,
                  embc_out, a_out, b_out, part_out):
    f32 = jnp.float32
    h0 = jnp.maximum(
        jnp.dot(x_ref[...], win_ref[...], preferred_element_type=f32)
        + bin_ref[...], 0.0)
    acc = jnp.zeros((N_NODES, HID), f32)
    csum_cols = jnp.zeros((N_NODES, 1), f32)
    for j in range(7):
        blk = (c2_ref[0, j * N_NODES:(j + 1) * N_NODES, :]
               + c2_ref[1, j * N_NODES:(j + 1) * N_NODES, :])
        acc = acc + jnp.dot(blk, h0[j * 128:(j + 1) * 128, :],
                            preferred_element_type=f32)
        csum_cols = csum_cols + jnp.sum(blk, axis=1, keepdims=True)
    deg = jnp.maximum(csum_cols, 1.0)
    agg = acc / deg
    reps = jnp.maximum(
        jnp.dot(agg, wmsg_ref[...], preferred_element_type=f32)
        + bmsg_ref[...], 0.0)
    emb_c = reps[:N_C]
    emb_g = reps[N_C:N_C + N_G]
    emb_p = reps[N_C + N_G:]
    dn = (((1,), (1,)), ((), ()))
    logits_r = jax.lax.dot_general(emb_g, emb_c, dn, preferred_element_type=f32)
    loss_rna = _kl_terms(_log_softmax(logits_r), trna_ref[...])
    logits_a = jax.lax.dot_general(emb_p, emb_c, dn, preferred_element_type=f32)
    loss_atac = _kl_terms(_log_softmax(logits_a), tatac_ref[...])
    logp_c = _log_softmax(emb_c)
    onehot = (jax.lax.broadcasted_iota(jnp.int32, (N_C, HID), 1)
              == lab_ref[...]).astype(f32)
    nll = -jnp.sum(logp_c * onehot, axis=1)
    smooth = -jnp.mean(logp_c, axis=1)
    loss_clust = jnp.mean((1.0 - EPS) * nll + EPS * smooth)
    part_out[...] = jnp.full((1, 1), loss_rna + loss_atac + loss_clust,
                             dtype=f32)
    embc_out[...] = emb_c
    a_out[...] = jnp.dot(emb_g, wdec_ref[:HID], preferred_element_type=f32)
    b_out[...] = (jnp.dot(emb_p, wdec_ref[HID:], preferred_element_type=f32)
                  + bdec_ref[...])


_stage_a_out_shape = (
    jax.ShapeDtypeStruct((N_C, HID), jnp.float32),
    jax.ShapeDtypeStruct((N_G, HID), jnp.float32),
    jax.ShapeDtypeStruct((N_P, HID), jnp.float32),
    jax.ShapeDtypeStruct((1, 1), jnp.float32),
)

_stage_a = pl.pallas_call(_stage_a_body, out_shape=_stage_a_out_shape)


_PB = 16
_NSTEP = N_P // _PB
_ROWS = _PB * N_G
_TOTAL = float(N_P * N_G * HID)


def _stage_b_body(tatac_ref, gp_ref, a_ref, trna_ref, embc_ref, b_ref,
                  part_ref, out_ref, lg_scr):
    i = pl.program_id(0)
    f32 = jnp.float32

    @pl.when(i == 0)
    def _():
        lg_scr[...] = jnp.log1p(trna_ref[...])
        out_ref[...] = part_ref[...]

    lg = lg_scr[...]
    lp_b = jnp.log1p(tatac_ref[...])
    w3 = lg[None, :, :] * lp_b[:, None, :]
    wr = w3.reshape(_ROWS, D_IN)
    cl = jnp.dot(wr, embc_ref[...], preferred_element_type=f32)
    scale = gp_ref[...] * (1.0 / 10000.0)
    cl = cl * scale
    m = jnp.max(cl, axis=-1, keepdims=True)
    e = jnp.exp(cl - m)
    ssum = jnp.sum(e, axis=-1, keepdims=True)
    pe3 = jnp.maximum(
        a_ref[...].reshape(_PB, 2, 1, HID) + b_ref[...][None, None], 0.0)
    pe = pe3.reshape(_ROWS, HID)
    pm = jnp.max(pe, axis=-1, keepdims=True)
    pu = jnp.exp(pe - pm)
    psum = jnp.sum(pu, axis=-1, keepdims=True)
    diff = (cl - m) - (pe - pm)
    num = jnp.sum(e * diff, axis=-1, keepdims=True)
    rowval = num / ssum - jnp.log(ssum) + jnp.log(psum)
    contrib = jnp.sum(rowval)
    out_ref[...] = out_ref[...] + contrib / _TOTAL


_stage_b_in_specs = [
    pl.BlockSpec((_PB, D_IN), lambda i: (i, 0)),
    pl.BlockSpec((_ROWS, 1), lambda i: (i, 0)),
    pl.BlockSpec((2 * _PB, HID), lambda i: (i, 0)),
    pl.BlockSpec((N_G, D_IN), lambda i: (0, 0)),
    pl.BlockSpec((N_C, HID), lambda i: (0, 0)),
    pl.BlockSpec((N_P, HID), lambda i: (0, 0)),
    pl.BlockSpec((1, 1), lambda i: (0, 0)),
]
_stage_b_out_spec = pl.BlockSpec((1, 1), lambda i: (0, 0))

_stage_b = pl.pallas_call(
    _stage_b_body,
    grid=(_NSTEP,),
    in_specs=_stage_b_in_specs,
    out_specs=_stage_b_out_spec,
    out_shape=jax.ShapeDtypeStruct((1, 1), jnp.float32),
    scratch_shapes=[pltpu.VMEM((N_G, D_IN), jnp.float32)],
)


def kernel(feat_c, feat_g, feat_p, target_rna, target_atac, gp_prior,
           W_in, b_in, W_msg, b_msg, W_dec, b_dec, edge_index, labels):
    c2 = _edge_hist()(edge_index)
    x = jnp.concatenate([feat_c, feat_g, feat_p], axis=0)
    emb_c, a_mat, b_mat, part = _stage_a(
        x, W_in, b_in.reshape(1, HID), c2, W_msg, b_msg.reshape(1, HID),
        target_rna, target_atac, labels.astype(jnp.int32).reshape(N_C, 1),
        W_dec, b_dec.reshape(1, HID))
    gp2 = gp_prior.reshape(N_G * N_P, 1)
    out = _stage_b(target_atac, gp2, a_mat, target_rna, emb_c, b_mat, part)
    return out[0, 0]


# speedup vs baseline: 1.0058x; 1.0058x over previous
import functools

import jax
import jax.numpy as jnp
from jax.experimental import pallas as pl
from jax.experimental.pallas import tpu as pltpu
from jax.experimental.pallas import tpu_sc as plsc

N_C, N_G, N_P = 512, 256, 128
D_IN, HID = 512, 256
N_NODES = N_C + N_G + N_P
N_EDGES = 131072
EPS = 0.1
NN2 = N_NODES * N_NODES

_SC_CORES = 2
_SC_SUBCORES = 16
_NW = _SC_CORES * _SC_SUBCORES
_EPW = N_EDGES // _NW
_CH = _EPW // 128
_ZPW = NN2 // _SC_SUBCORES


def _edge_hist_body(edge_hbm, out_hbm, src_v, dst_v, idx_v, ones_v,
                    zbuf_a, zbuf_b, stage_a, stage_b,
                    sem, sem_i0, sem_i1, sem_o0, sem_o1, csh):
    c = jax.lax.axis_index("c")
    s = jax.lax.axis_index("s")
    wid = s * _SC_CORES + c
    base = wid * _EPW
    cp_s = pltpu.async_copy(edge_hbm.at[0, pl.ds(base, _EPW)], src_v, sem)
    cp_d = pltpu.async_copy(edge_hbm.at[1, pl.ds(base, _EPW)], dst_v, sem)
    for k in range(_ZBUF // 16):
        zbuf_a[pl.ds(k * 16, 16)] = jnp.zeros((16,), jnp.float32)
    zcopies = [
        pltpu.async_copy(zbuf_a, csh.at[pl.ds(s * _ZPW + q * _ZBUF, _ZBUF)],
                         sem_i0)
        for q in range(_ZPW // _ZBUF)
    ]
    for k in range(8):
        ones_v[pl.ds(k * 16, 16)] = jnp.ones((16,), jnp.float32)
    cp_s.wait()
    cp_d.wait()
    for j in range(_CH):
        for k in range(8):
            o = (j * 8 + k) * 16
            srcs = src_v[pl.ds(o, 16)]
            dsts = dst_v[pl.ds(o, 16)]
            idx_v[j, pl.ds(k * 16, 16)] = (
                (srcs >> 7) * (N_NODES * 128) + dsts * 128 + (srcs & 127))
    for cp in zcopies:
        cp.wait()
    plsc.subcore_barrier()
    copies = [
        pltpu.async_copy(ones_v, csh.at[idx_v.at[j]], sem, add=True)
        for j in range(_CH)
    ]
    for cp in copies:
        cp.wait()
    plsc.subcore_barrier()
    crows = _ZBUF // 128
    nch = _ZPW // _ZBUF
    zbufs = [zbuf_a, zbuf_b]
    stages = [stage_a, stage_b]
    sem_in = [sem_i0, sem_i1]
    sem_out = [sem_o0, sem_o1]
    in_cp = [None, None]
    out_cp = [None, None]

    def _chunk_src(q):
        return csh.at[pl.ds(s * _ZPW + q * _ZBUF, _ZBUF)]

    in_cp[0] = pltpu.async_copy(_chunk_src(0), zbuf_a, sem_i0)
    for q in range(nch):
        b = q & 1
        in_cp[b].wait()
        if q + 1 < nch:
            in_cp[1 - b] = pltpu.async_copy(_chunk_src(q + 1), zbufs[1 - b],
                                            sem_in[1 - b])
        if out_cp[b] is not None:
            out_cp[b].wait()
        zb = zbufs[b]
        st = stages[b]

        def _row(r, carry):
            for k in range(8):
                st[r, pl.ds(k * 16, 16)] = zb[pl.ds(r * 128 + k * 16, 16)]
            return carry

        jax.lax.fori_loop(0, crows, _row, 0)
        out_cp[b] = pltpu.async_copy(
            st, out_hbm.at[c, pl.ds(s * (_ZPW // 128) + q * crows, crows), :],
            sem_out[b])
    out_cp[0].wait()
    out_cp[1].wait()


_ZBUF = 7168


@functools.cache
def _edge_hist():
    return pl.kernel(
        _edge_hist_body,
        out_type=jax.ShapeDtypeStruct((_SC_CORES, NN2 // 128, 128),
                                      jnp.float32),
        mesh=plsc.VectorSubcoreMesh(core_axis_name="c", subcore_axis_name="s",
                                    num_cores=_SC_CORES,
                                    num_subcores=_SC_SUBCORES),
        scratch_types=[
            pltpu.VMEM((_EPW,), jnp.int32),
            pltpu.VMEM((_EPW,), jnp.int32),
            pltpu.VMEM((_CH, 128), jnp.int32),
            pltpu.VMEM((128,), jnp.float32),
            pltpu.VMEM((_ZBUF,), jnp.float32),
            pltpu.VMEM((_ZBUF,), jnp.float32),
            pltpu.VMEM((_ZBUF // 128, 128), jnp.float32),
            pltpu.VMEM((_ZBUF // 128, 128), jnp.float32),
            pltpu.SemaphoreType.DMA,
            pltpu.SemaphoreType.DMA,
            pltpu.SemaphoreType.DMA,
            pltpu.SemaphoreType.DMA,
            pltpu.SemaphoreType.DMA,
            pltpu.VMEM_SHARED((NN2,), jnp.float32),
        ],
    )


def _kl_terms(log_pred, target_logits):
    m = jnp.max(target_logits, axis=-1, keepdims=True)
    e = jnp.exp(target_logits - m)
    ssum = jnp.sum(e, axis=-1, keepdims=True)
    t = e / ssum
    log_t = (target_logits - m) - jnp.log(ssum)
    return jnp.mean(t * (log_t - log_pred))


def _log_softmax(z):
    m = jnp.max(z, axis=-1, keepdims=True)
    zs = z - m
    return zs - jnp.log(jnp.sum(jnp.exp(zs), axis=-1, keepdims=True))


def _stage_a_body(x_ref, win_ref, bin_ref, c2_ref, wmsg_ref, bmsg_ref,
                  trna_ref, tatac_ref, lab_ref, wdec_ref, bdec_ref,
                  embc_out, a_out, b_out, part_out, csum_scr):
    f32 = jnp.float32
    h0 = jnp.maximum(
        jnp.dot(x_ref[...], win_ref[...], preferred_element_type=f32)
        + bin_ref[...], 0.0)
    for j in range(7):
        csum_scr[:, j * 128:(j + 1) * 128] = (
            c2_ref[0, j * N_NODES:(j + 1) * N_NODES, :]
            + c2_ref[1, j * N_NODES:(j + 1) * N_NODES, :])
    csum = csum_scr[...]
    acc = jnp.dot(csum, h0, preferred_element_type=f32)
    deg = jnp.maximum(jnp.sum(csum, axis=1, keepdims=True), 1.0)
    agg = acc / deg
    reps = jnp.maximum(
        jnp.dot(agg, wmsg_ref[...], preferred_element_type=f32)
        + bmsg_ref[...], 0.0)
    emb_c = reps[:N_C]
    emb_g = reps[N_C:N_C + N_G]
    emb_p = reps[N_C + N_G:]
    dn = (((1,), (1,)), ((), ()))
    logits_r = jax.lax.dot_general(emb_g, emb_c, dn, preferred_element_type=f32)
    loss_rna = _kl_terms(_log_softmax(logits_r), trna_ref[...])
    logits_a = jax.lax.dot_general(emb_p, emb_c, dn, preferred_element_type=f32)
    loss_atac = _kl_terms(_log_softmax(logits_a), tatac_ref[...])
    logp_c = _log_softmax(emb_c)
    onehot = (jax.lax.broadcasted_iota(jnp.int32, (N_C, HID), 1)
              == lab_ref[...]).astype(f32)
    nll = -jnp.sum(logp_c * onehot, axis=1)
    smooth = -jnp.mean(logp_c, axis=1)
    loss_clust = jnp.mean((1.0 - EPS) * nll + EPS * smooth)
    part_out[...] = jnp.full((1, 1), loss_rna + loss_atac + loss_clust,
                             dtype=f32)
    embc_out[...] = emb_c
    a_out[...] = jnp.dot(emb_g, wdec_ref[:HID], preferred_element_type=f32)
    b_out[...] = (jnp.dot(emb_p, wdec_ref[HID:], preferred_element_type=f32)
                  + bdec_ref[...])


_stage_a_out_shape = (
    jax.ShapeDtypeStruct((N_C, HID), jnp.float32),
    jax.ShapeDtypeStruct((N_G, HID), jnp.float32),
    jax.ShapeDtypeStruct((N_P, HID), jnp.float32),
    jax.ShapeDtypeStruct((1, 1), jnp.float32),
)

_stage_a = pl.pallas_call(
    _stage_a_body, out_shape=_stage_a_out_shape,
    scratch_shapes=[pltpu.VMEM((N_NODES, N_NODES), jnp.float32)])


_PB = 8
_NSTEP = N_P // _PB
_ROWS = _PB * N_G
_TOTAL = float(N_P * N_G * HID)


def _stage_b_body(tatac_ref, gp_ref, a_ref, trna_ref, embc_ref, b_ref,
                  part_ref, out_ref, lg_scr):
    i = pl.program_id(0)
    f32 = jnp.float32

    @pl.when(i == 0)
    def _():
        lg_scr[...] = jnp.log1p(trna_ref[...])
        out_ref[...] = part_ref[...]

    lg = lg_scr[...]
    lp_b = jnp.log1p(tatac_ref[...])
    w3 = lg[None, :, :] * lp_b[:, None, :]
    wr = w3.reshape(_ROWS, D_IN)
    cl = jnp.dot(wr, embc_ref[...], preferred_element_type=f32)
    scale = gp_ref[...] * (1.0 / 10000.0)
    cl = cl * scale
    m = jnp.max(cl, axis=-1, keepdims=True)
    e = jnp.exp(cl - m)
    ssum = jnp.sum(e, axis=-1, keepdims=True)
    pe3 = jnp.maximum(
        a_ref[...].reshape(_PB, 2, 1, HID) + b_ref[...][None, None], 0.0)
    pe = pe3.reshape(_ROWS, HID)
    pm = jnp.max(pe, axis=-1, keepdims=True)
    pu = jnp.exp(pe - pm)
    psum = jnp.sum(pu, axis=-1, keepdims=True)
    diff = (cl - m) - (pe - pm)
    num = jnp.sum(e * diff, axis=-1, keepdims=True)
    rowval = num / ssum - jnp.log(ssum) + jnp.log(psum)
    contrib = jnp.sum(rowval)
    out_ref[...] = out_ref[...] + contrib / _TOTAL


_stage_b_in_specs = [
    pl.BlockSpec((_PB, D_IN), lambda i: (i, 0)),
    pl.BlockSpec((_ROWS, 1), lambda i: (i, 0)),
    pl.BlockSpec((2 * _PB, HID), lambda i: (i, 0)),
    pl.BlockSpec((N_G, D_IN), lambda i: (0, 0)),
    pl.BlockSpec((N_C, HID), lambda i: (0, 0)),
    pl.BlockSpec((N_P, HID), lambda i: (0, 0)),
    pl.BlockSpec((1, 1), lambda i: (0, 0)),
]
_stage_b_out_spec = pl.BlockSpec((1, 1), lambda i: (0, 0))

_stage_b = pl.pallas_call(
    _stage_b_body,
    grid=(_NSTEP,),
    in_specs=_stage_b_in_specs,
    out_specs=_stage_b_out_spec,
    out_shape=jax.ShapeDtypeStruct((1, 1), jnp.float32),
    scratch_shapes=[pltpu.VMEM((N_G, D_IN), jnp.float32)],
)


def kernel(feat_c, feat_g, feat_p, target_rna, target_atac, gp_prior,
           W_in, b_in, W_msg, b_msg, W_dec, b_dec, edge_index, labels):
    c2 = _edge_hist()(edge_index)
    x = jnp.concatenate([feat_c, feat_g, feat_p], axis=0)
    emb_c, a_mat, b_mat, part = _stage_a(
        x, W_in, b_in.reshape(1, HID), c2, W_msg, b_msg.reshape(1, HID),
        target_rna, target_atac, labels.astype(jnp.int32).reshape(N_C, 1),
        W_dec, b_dec.reshape(1, HID))
    gp2 = gp_prior.reshape(N_G * N_P, 1)
    out = _stage_b(target_atac, gp2, a_mat, target_rna, emb_c, b_mat, part)
    return out[0, 0]
